# Initial kernel scaffold; baseline (speedup 1.0000x reference)
import jax
import jax.numpy as jnp
from jax.experimental import pallas as pl


def kernel(x, edge_index, edge_weight, Wf, bf, Wg, bg, Wskip, Wgnn, gamma, beta):
    raise NotImplementedError("write your pallas kernel here")


# speedup vs baseline: 3.4687x; 3.4687x over previous
import functools

import jax
import jax.numpy as jnp
from jax import lax
from jax.experimental import pallas as pl
from jax.experimental.pallas import tpu as pltpu
from jax.experimental.pallas import tpu_sc as plsc

N = 10000
E = 160000
C = 128
T = 8
NT = N * T
TC = T * C
L = 3
EPS = 1e-5

NTILES = 32
EW = E // NTILES
K = 32
VCH = TC // 16
ZR = 8

RB = 1600
NB = NT // RB

def _sc_spmm(h_nodes, src_s, dst_s, w_s, bounds):
    _mesh = plsc.VectorSubcoreMesh(
        core_axis_name="c", subcore_axis_name="s", num_cores=2, num_subcores=16
    )

    @functools.partial(
        pl.kernel,
        out_type=jax.ShapeDtypeStruct((N, TC), jnp.float32),
        mesh=_mesh,
        scratch_types=[
            pltpu.VMEM((K,), jnp.int32),
            pltpu.VMEM((K, TC), jnp.float32),
            pltpu.VMEM((ZR, TC), jnp.float32),
            pltpu.VMEM((ZR, TC), jnp.float32),
            pltpu.VMEM((NTILES, 16), jnp.int32),
            pltpu.VMEM((K + 16,), jnp.int32),
            pltpu.VMEM((K + 16,), jnp.float32),
        ],
    )
    def kern(h_hbm, src_hbm, dst_hbm, w_hbm, b_hbm, agg_hbm,
             idx_v, rows_v, stg_v, zero_v, b_sm, dst_sm, w_sm):
        wid = lax.axis_index("s") * 2 + lax.axis_index("c")
        pltpu.sync_copy(b_hbm, b_sm)
        brow = b_sm[wid, pl.ds(0, 16)]
        nlo = brow[0]
        nhi = brow[1]
        est = brow[2]
        een = brow[3]

        zvec = jnp.zeros((16,), jnp.float32)

        @pl.loop(0, ZR)
        def _(r):
            @pl.loop(0, VCH)
            def _(cc):
                zero_v[r, pl.ds(cc * 16, 16)] = zvec
                stg_v[r, pl.ds(cc * 16, 16)] = zvec

        @pl.when(nlo < nhi)
        def _():
            @pl.loop(nlo, nhi, step=ZR)
            def _(n):
                pltpu.sync_copy(zero_v, agg_hbm.at[pl.ds(pl.multiple_of(n, ZR), ZR)])

            base0 = (est // 8) * 8
            nch = (een - base0 + (K - 1)) // K

            def chunk_body(cidx, cur_g):
                base = base0 + cidx * K
                pltpu.sync_copy(src_hbm.at[pl.ds(base, K)], idx_v)
                pltpu.sync_copy(dst_hbm.at[pl.ds(base, K)], dst_sm.at[pl.ds(0, K)])
                pltpu.sync_copy(w_hbm.at[pl.ds(base, K)], w_sm.at[pl.ds(0, K)])
                pltpu.sync_copy(h_hbm.at[idx_v], rows_v)

                def edge_body(j, cur_g_in):
                    e = base + j
                    valid = jnp.logical_and(e >= est, e < een)
                    wj = jnp.where(valid, w_sm[pl.ds(j, 16)][0], 0.0)
                    d = jnp.where(valid, dst_sm[pl.ds(j, 16)][0], cur_g_in * ZR)
                    d = jnp.clip(d, nlo, nhi - 1)
                    g = d // ZR

                    @pl.when(g != cur_g_in)
                    def _():
                        pltpu.sync_copy(
                            stg_v,
                            agg_hbm.at[pl.ds(pl.multiple_of(cur_g_in * ZR, ZR), ZR)])

                        @pl.loop(0, ZR)
                        def _(r):
                            @pl.loop(0, VCH)
                            def _(cc):
                                stg_v[r, pl.ds(cc * 16, 16)] = zvec

                    @pl.when(wj != 0.0)
                    def _():
                        r = d - g * ZR

                        @pl.loop(0, VCH, unroll=8)
                        def _(cc):
                            sl = pl.ds(cc * 16, 16)
                            plsc.addupdate(stg_v.at[r, sl], wj * rows_v[j, sl])

                    return g

                return pl.loop(0, K, init_carry=cur_g)(edge_body)

            cur_g_fin = pl.loop(0, nch, init_carry=nlo // ZR)(chunk_body)
            pltpu.sync_copy(
                stg_v, agg_hbm.at[pl.ds(pl.multiple_of(cur_g_fin * ZR, ZR), ZR)])

    return kern(h_nodes, src_s, dst_s, w_s, bounds)


def _tc_gate(X, Sin, Wtap, Wx, bvec, WskipT, d):
    def body(x_ref, s_ref, wt_ref, wx_ref, b_ref, wsk_ref, h_ref, so_ref):
        xb = x_ref[...]
        sh = jnp.concatenate(
            [jnp.zeros((d, C), jnp.float32), xb[: RB - d, :]], axis=0)
        tmask = (lax.broadcasted_iota(jnp.int32, (RB, 1), 0) % T) >= d
        tap = jnp.where(tmask, sh, 0.0)
        fg = (jnp.dot(tap, wt_ref[...], preferred_element_type=jnp.float32)
              + jnp.dot(xb, wx_ref[...], preferred_element_type=jnp.float32)
              + b_ref[...])
        h = jnp.tanh(fg[:, :C]) * jax.nn.sigmoid(fg[:, C:])
        h_ref[...] = h
        so_ref[...] = s_ref[...] + jnp.dot(
            h, wsk_ref[...], preferred_element_type=jnp.float32)

    return pl.pallas_call(
        body,
        grid=(NB,),
        in_specs=[
            pl.BlockSpec((RB, C), lambda i: (i, 0)),
            pl.BlockSpec((RB, C), lambda i: (i, 0)),
            pl.BlockSpec((C, 2 * C), lambda i: (0, 0)),
            pl.BlockSpec((C, 2 * C), lambda i: (0, 0)),
            pl.BlockSpec((1, 2 * C), lambda i: (0, 0)),
            pl.BlockSpec((C, C), lambda i: (0, 0)),
        ],
        out_specs=[
            pl.BlockSpec((RB, C), lambda i: (i, 0)),
            pl.BlockSpec((RB, C), lambda i: (i, 0)),
        ],
        out_shape=[
            jax.ShapeDtypeStruct((NT, C), jnp.float32),
            jax.ShapeDtypeStruct((NT, C), jnp.float32),
        ],
    )(X, Sin, Wtap, Wx, bvec, WskipT)


def _tc_gnn_bn(AGG, X, WgT, gb):
    def body(agg_ref, x_ref, wg_ref, gb_ref, o_ref, sum_ref, ssq_ref):
        ph = pl.program_id(0)
        j = pl.program_id(1)
        v = x_ref[...] + jnp.dot(
            agg_ref[...], wg_ref[...], preferred_element_type=jnp.float32)

        @pl.when(jnp.logical_and(ph == 0, j == 0))
        def _():
            sum_ref[...] = jnp.zeros_like(sum_ref)
            ssq_ref[...] = jnp.zeros_like(ssq_ref)

        @pl.when(ph == 0)
        def _():
            sum_ref[...] += jnp.sum(v, axis=0, keepdims=True)
            ssq_ref[...] += jnp.sum(v * v, axis=0, keepdims=True)

        @pl.when(ph == 1)
        def _():
            mean = sum_ref[...] * (1.0 / NT)
            var = ssq_ref[...] * (1.0 / NT) - mean * mean
            scale = gb_ref[0:1, :] * lax.rsqrt(var + EPS)
            o_ref[...] = scale * (v - mean) + gb_ref[1:2, :]

    return pl.pallas_call(
        body,
        grid=(2, NB),
        in_specs=[
            pl.BlockSpec((RB, C), lambda p, i: (i, 0)),
            pl.BlockSpec((RB, C), lambda p, i: (i, 0)),
            pl.BlockSpec((C, C), lambda p, i: (0, 0)),
            pl.BlockSpec((2, C), lambda p, i: (0, 0)),
        ],
        out_specs=pl.BlockSpec((RB, C), lambda p, i: (i, 0)),
        out_shape=jax.ShapeDtypeStruct((NT, C), jnp.float32),
        scratch_shapes=[
            pltpu.VMEM((1, C), jnp.float32),
            pltpu.VMEM((1, C), jnp.float32),
        ],
    )(AGG, X, WgT, gb)


def kernel(x, edge_index, edge_weight, Wf, bf, Wg, bg, Wskip, Wgnn, gamma, beta):
    X = x[0].transpose(1, 2, 0).reshape(NT, C)

    src = edge_index[0].astype(jnp.int32)
    dst = edge_index[1].astype(jnp.int32)
    order = jnp.argsort(dst)
    src_s = src[order]
    dst_s = dst[order]
    w_s = edge_weight[order]
    rp = jnp.searchsorted(
        dst_s, jnp.arange(N + 1, dtype=jnp.int32), side="left").astype(jnp.int32)
    cuts = jnp.arange(NTILES, dtype=jnp.int32) * EW
    nlo = jnp.searchsorted(rp, cuts, side="left").astype(jnp.int32)
    nlo = (nlo // ZR) * ZR
    nlo_ext = jnp.concatenate([nlo, jnp.array([N], jnp.int32)])
    estart = rp[nlo_ext[:NTILES]]
    eend = rp[nlo_ext[1:]]
    bounds = jnp.zeros((NTILES, 16), jnp.int32)
    bounds = bounds.at[:, 0].set(nlo_ext[:NTILES])
    bounds = bounds.at[:, 1].set(nlo_ext[1:])
    bounds = bounds.at[:, 2].set(estart)
    bounds = bounds.at[:, 3].set(eend)
    src_p = jnp.concatenate([src_s, jnp.zeros((K,), jnp.int32)])
    dst_p = jnp.concatenate([dst_s, jnp.full((K,), N - 1, jnp.int32)])
    w_p = jnp.concatenate([w_s, jnp.zeros((K,), jnp.float32)])

    S = jnp.zeros((NT, C), jnp.float32)
    for l in range(L):
        Wtap = jnp.concatenate([Wf[l, :, :, 0].T, Wg[l, :, :, 0].T], axis=1)
        Wx = jnp.concatenate([Wf[l, :, :, 1].T, Wg[l, :, :, 1].T], axis=1)
        bvec = jnp.concatenate([bf[l], bg[l]]).reshape(1, 2 * C)
        H, S = _tc_gate(X, S, Wtap, Wx, bvec, Wskip[l].T, 2 ** l)
        AGG = _sc_spmm(H.reshape(N, TC), src_p, dst_p, w_p, bounds)
        gbmat = jnp.stack([gamma[l], beta[l]], axis=0)
        X = _tc_gnn_bn(AGG.reshape(NT, C), X, Wgnn[l].T, gbmat)

    x_out = X.reshape(N, T, C).transpose(2, 0, 1)[None]
    s_out = S.reshape(N, T, C).transpose(2, 0, 1)[None]
    return (x_out, s_out)
